# baseline (speedup 1.0000x reference)
import jax
import jax.numpy as jnp
from jax.experimental import pallas as pl

_B, _S, _D = 1024, 128, 768
_POOL = 4096
_TOP_K = 16

_EB = 32
_BB = 64


def _mean_kernel(x_ref, o_ref):
    o_ref[...] = jnp.sum(x_ref[...], axis=1) * (1.0 / _S)


def _l2n(x):
    ss = jnp.sum(x * x, axis=-1, keepdims=True)
    return x * jax.lax.rsqrt(jnp.maximum(ss, 1e-12))


def _sim_topk_kernel(xn_ref, k0_ref, k1_ref, k2_ref,
                     s0_ref, v0_ref, i0_ref,
                     s1_ref, v1_ref, i1_ref,
                     s2_ref, v2_ref, i2_ref):
    xn = xn_ref[...]
    iota = jax.lax.broadcasted_iota(jnp.int32, (_BB, _POOL), 1)
    for k_ref, s_ref, v_ref, i_ref in (
            (k0_ref, s0_ref, v0_ref, i0_ref),
            (k1_ref, s1_ref, v1_ref, i1_ref),
            (k2_ref, s2_ref, v2_ref, i2_ref)):
        sim = jax.lax.dot_general(
            xn, k_ref[...],
            dimension_numbers=(((1,), (1,)), ((), ())),
            preferred_element_type=jnp.float32)
        s_ref[...] = sim

        w = sim
        vals, idxs = [], []
        for _ in range(_TOP_K):
            m = jnp.max(w, axis=1)
            idx = jnp.min(jnp.where(w == m[:, None], iota, _POOL), axis=1)
            vals.append(m)
            idxs.append(idx)
            w = jnp.where(iota == idx[:, None], -jnp.inf, w)
        v_ref[...] = jnp.stack(vals, axis=1)
        i_ref[...] = jnp.stack(idxs, axis=1)


def kernel(x_embed, s_prompt_key, t_prompt_key, m_prompt_key):
    xm = pl.pallas_call(
        _mean_kernel,
        grid=(_B // _EB,),
        in_specs=[pl.BlockSpec((_EB, _S, _D), lambda i: (i, 0, 0))],
        out_specs=pl.BlockSpec((_EB, _D), lambda i: (i, 0)),
        out_shape=jax.ShapeDtypeStruct((_B, _D), jnp.float32),
    )(x_embed)

    xn = _l2n(xm)
    kn = [_l2n(k) for k in (s_prompt_key, t_prompt_key, m_prompt_key)]

    out = pl.pallas_call(
        _sim_topk_kernel,
        grid=(_B // _BB,),
        in_specs=[
            pl.BlockSpec((_BB, _D), lambda i: (i, 0)),
            pl.BlockSpec((_POOL, _D), lambda i: (0, 0)),
            pl.BlockSpec((_POOL, _D), lambda i: (0, 0)),
            pl.BlockSpec((_POOL, _D), lambda i: (0, 0)),
        ],
        out_specs=[spec for _ in range(3) for spec in (
            pl.BlockSpec((_BB, _POOL), lambda i: (i, 0)),
            pl.BlockSpec((_BB, _TOP_K), lambda i: (i, 0)),
            pl.BlockSpec((_BB, _TOP_K), lambda i: (i, 0)),
        )],
        out_shape=[shape for _ in range(3) for shape in (
            jax.ShapeDtypeStruct((_B, _POOL), jnp.float32),
            jax.ShapeDtypeStruct((_B, _TOP_K), jnp.float32),
            jax.ShapeDtypeStruct((_B, _TOP_K), jnp.int32),
        )],
    )(xn, *kn)
    return tuple(out)


# speedup vs baseline: 1.1780x; 1.1780x over previous
import jax
import jax.numpy as jnp
from jax.experimental import pallas as pl
from jax.experimental.pallas import tpu as pltpu

_B, _S, _D = 1024, 128, 768
_POOL = 4096
_TOP_K = 16

_EB = 32
_BB = 128


def _mean_kernel(x_ref, o_ref):
    o_ref[...] = jnp.sum(x_ref[...], axis=1) * (1.0 / _S)


def _inv_norm(v):
    ss = jnp.sum(v * v, axis=-1, keepdims=True)
    return jax.lax.rsqrt(jnp.maximum(ss, 1e-12))


def _sim_topk_kernel(xm_ref, xs_ref, key_ref, ks_ref, sim_ref, topv_ref,
                     topi_ref, kn_ref):
    kn_ref[...] = key_ref[...] * ks_ref[...]

    xn = xm_ref[...] * xs_ref[...]
    sim = jax.lax.dot_general(
        xn, kn_ref[...],
        dimension_numbers=(((1,), (1,)), ((), ())),
        preferred_element_type=jnp.float32)
    sim_ref[...] = sim

    w = sim
    iota = jax.lax.broadcasted_iota(jnp.int32, sim.shape, 1)
    liota = jax.lax.broadcasted_iota(jnp.int32, (_BB, 128), 1)
    vals, idxs = [], []
    for _ in range(_TOP_K):
        m = jnp.max(w, axis=1)
        mb = m[:, None]
        cmin = jnp.full((_BB, 128), 32, jnp.int32)
        for c in range(31, -1, -1):
            cmin = jnp.where(w[:, c * 128:(c + 1) * 128] == mb, c, cmin)
        idx = jnp.min(jnp.where(cmin < 32, cmin * 128 + liota, _POOL), axis=1)
        vals.append(m)
        idxs.append(idx)
        w = jnp.where(iota == idx[:, None], -jnp.inf, w)
    topv_ref[...] = jnp.stack(vals, axis=1)
    topi_ref[...] = jnp.stack(idxs, axis=1)


def _pool_sim_topk(xm, xs, key, ks):
    return pl.pallas_call(
        _sim_topk_kernel,
        grid=(_B // _BB,),
        in_specs=[
            pl.BlockSpec((_BB, _D), lambda i: (i, 0)),
            pl.BlockSpec((_BB, 1), lambda i: (i, 0)),
            pl.BlockSpec((_POOL, _D), lambda i: (0, 0)),
            pl.BlockSpec((_POOL, 1), lambda i: (0, 0)),
        ],
        out_specs=[
            pl.BlockSpec((_BB, _POOL), lambda i: (i, 0)),
            pl.BlockSpec((_BB, _TOP_K), lambda i: (i, 0)),
            pl.BlockSpec((_BB, _TOP_K), lambda i: (i, 0)),
        ],
        out_shape=[
            jax.ShapeDtypeStruct((_B, _POOL), jnp.float32),
            jax.ShapeDtypeStruct((_B, _TOP_K), jnp.float32),
            jax.ShapeDtypeStruct((_B, _TOP_K), jnp.int32),
        ],
        scratch_shapes=[pltpu.VMEM((_POOL, _D), jnp.float32)],
        compiler_params=pltpu.CompilerParams(
            dimension_semantics=("parallel",)),
    )(xm, xs, key, ks)


def kernel(x_embed, s_prompt_key, t_prompt_key, m_prompt_key):
    xm = pl.pallas_call(
        _mean_kernel,
        grid=(_B // _EB,),
        in_specs=[pl.BlockSpec((_EB, _S, _D), lambda i: (i, 0, 0))],
        out_specs=pl.BlockSpec((_EB, _D), lambda i: (i, 0)),
        out_shape=jax.ShapeDtypeStruct((_B, _D), jnp.float32),
        compiler_params=pltpu.CompilerParams(
            dimension_semantics=("parallel",)),
    )(x_embed)

    xs = _inv_norm(xm)
    s_sim, s_v, s_i = _pool_sim_topk(xm, xs, s_prompt_key, _inv_norm(s_prompt_key))
    t_sim, t_v, t_i = _pool_sim_topk(xm, xs, t_prompt_key, _inv_norm(t_prompt_key))
    m_sim, m_v, m_i = _pool_sim_topk(xm, xs, m_prompt_key, _inv_norm(m_prompt_key))
    return (s_sim, s_v, s_i, t_sim, t_v, t_i, m_sim, m_v, m_i)


# speedup vs baseline: 1.3788x; 1.1704x over previous
import jax
import jax.numpy as jnp
from jax.experimental import pallas as pl
from jax.experimental.pallas import tpu as pltpu

_B, _S, _D = 1024, 128, 768
_POOL = 4096
_TOP_K = 16

_EB = 32
_BB = 256


def _mean_kernel(x_ref, o_ref):
    o_ref[...] = jnp.sum(x_ref[...], axis=1) * (1.0 / _S)


def _inv_norm(v):
    ss = jnp.sum(v * v, axis=-1, keepdims=True)
    return jax.lax.rsqrt(jnp.maximum(ss, 1e-12))


def _sim_topk_kernel(xm_ref, xs_ref, key_ref, ks_ref, sim_ref, topv_ref,
                     topi_ref, kn_ref):
    kn_ref[...] = key_ref[...] * ks_ref[...]

    xn = xm_ref[...] * xs_ref[...]
    sim = jax.lax.dot_general(
        xn, kn_ref[...],
        dimension_numbers=(((1,), (1,)), ((), ())),
        preferred_element_type=jnp.float32)
    sim_ref[...] = sim

    w = sim
    iota = jax.lax.broadcasted_iota(jnp.int32, sim.shape, 1)
    liota = jax.lax.broadcasted_iota(jnp.int32, (_BB, 128), 1)
    vals, idxs = [], []
    for _ in range(_TOP_K):
        m = jnp.max(w, axis=1)
        mb = m[:, None]
        cmin = jnp.full((_BB, 128), 32, jnp.int32)
        for c in range(31, -1, -1):
            cmin = jnp.where(w[:, c * 128:(c + 1) * 128] == mb, c, cmin)
        idx = jnp.min(jnp.where(cmin < 32, cmin * 128 + liota, _POOL), axis=1)
        vals.append(m)
        idxs.append(idx)
        w = jnp.where(iota == idx[:, None], -jnp.inf, w)
    topv_ref[...] = jnp.stack(vals, axis=1)
    topi_ref[...] = jnp.stack(idxs, axis=1)


def _pool_sim_topk(xm, xs, key, ks):
    return pl.pallas_call(
        _sim_topk_kernel,
        grid=(_B // _BB,),
        in_specs=[
            pl.BlockSpec((_BB, _D), lambda i: (i, 0)),
            pl.BlockSpec((_BB, 1), lambda i: (i, 0)),
            pl.BlockSpec((_POOL, _D), lambda i: (0, 0)),
            pl.BlockSpec((_POOL, 1), lambda i: (0, 0)),
        ],
        out_specs=[
            pl.BlockSpec((_BB, _POOL), lambda i: (i, 0)),
            pl.BlockSpec((_BB, _TOP_K), lambda i: (i, 0)),
            pl.BlockSpec((_BB, _TOP_K), lambda i: (i, 0)),
        ],
        out_shape=[
            jax.ShapeDtypeStruct((_B, _POOL), jnp.float32),
            jax.ShapeDtypeStruct((_B, _TOP_K), jnp.float32),
            jax.ShapeDtypeStruct((_B, _TOP_K), jnp.int32),
        ],
        scratch_shapes=[pltpu.VMEM((_POOL, _D), jnp.float32)],
        compiler_params=pltpu.CompilerParams(
            dimension_semantics=("parallel",)),
    )(xm, xs, key, ks)


def kernel(x_embed, s_prompt_key, t_prompt_key, m_prompt_key):
    xm = pl.pallas_call(
        _mean_kernel,
        grid=(_B // _EB,),
        in_specs=[pl.BlockSpec((_EB, _S, _D), lambda i: (i, 0, 0))],
        out_specs=pl.BlockSpec((_EB, _D), lambda i: (i, 0)),
        out_shape=jax.ShapeDtypeStruct((_B, _D), jnp.float32),
        compiler_params=pltpu.CompilerParams(
            dimension_semantics=("parallel",)),
    )(x_embed)

    xs = _inv_norm(xm)
    s_sim, s_v, s_i = _pool_sim_topk(xm, xs, s_prompt_key, _inv_norm(s_prompt_key))
    t_sim, t_v, t_i = _pool_sim_topk(xm, xs, t_prompt_key, _inv_norm(t_prompt_key))
    m_sim, m_v, m_i = _pool_sim_topk(xm, xs, m_prompt_key, _inv_norm(m_prompt_key))
    return (s_sim, s_v, s_i, t_sim, t_v, t_i, m_sim, m_v, m_i)
